# Initial kernel scaffold; baseline (speedup 1.0000x reference)
import jax
import jax.numpy as jnp
from jax.experimental import pallas as pl


def kernel(x, pos, batch, params):
    raise NotImplementedError("write your pallas kernel here")


# speedup vs baseline: 8.2933x; 8.2933x over previous
import functools

import jax
import jax.numpy as jnp
from jax import lax
from jax.experimental import pallas as pl
from jax.experimental.pallas import tpu as pltpu
from jax.experimental.pallas import tpu_sc as plsc

N = 4096
KNN = 16
BIG = 1e9
EPS = 1e-5
_KNN_ROWS = 256
_CONV_ROWS = 256


def _rowsq_body(x_ref, sq_ref):
    xv = x_ref[...]
    sq_ref[...] = jnp.sum(xv * xv, axis=1, keepdims=True)


def _rowsq(xfeat):
    return pl.pallas_call(
        _rowsq_body,
        out_shape=jax.ShapeDtypeStruct((N, 1), jnp.float32),
    )(xfeat)


def _knn_body(xr_ref, xT_ref, sqr_ref, sqc_ref, br_ref, bc_ref, idx_ref, *, rows):
    i = pl.program_id(0)
    prod = jnp.dot(xr_ref[...], xT_ref[...], preferred_element_type=jnp.float32)
    score = (sqr_ref[...] + sqc_ref[...]) - 2.0 * prod
    row_ids = i * rows + lax.broadcasted_iota(jnp.int32, (rows, N), 0)
    col_ids = lax.broadcasted_iota(jnp.int32, (rows, N), 1)
    score = jnp.where(br_ref[...] != bc_ref[...], BIG, score)
    score = score + jnp.where(row_ids == col_ids, BIG, 0.0)
    for t in range(KNN):
        m = jnp.min(score, axis=1, keepdims=True)
        amin = jnp.min(jnp.where(score == m, col_ids, N), axis=1, keepdims=True)
        idx_ref[:, pl.ds(t, 1)] = amin
        score = jnp.where(col_ids == amin, BIG, score)


def _knn(xfeat, sq, batch):
    d = xfeat.shape[1]
    r = _KNN_ROWS
    return pl.pallas_call(
        functools.partial(_knn_body, rows=r),
        grid=(N // r,),
        in_specs=[
            pl.BlockSpec((r, d), lambda i: (i, 0)),
            pl.BlockSpec((d, N), lambda i: (0, 0)),
            pl.BlockSpec((r, 1), lambda i: (i, 0)),
            pl.BlockSpec((1, N), lambda i: (0, 0)),
            pl.BlockSpec((r, 1), lambda i: (i, 0)),
            pl.BlockSpec((1, N), lambda i: (0, 0)),
        ],
        out_specs=pl.BlockSpec((r, KNN), lambda i: (i, 0)),
        out_shape=jax.ShapeDtypeStruct((N, KNN), jnp.int32),
    )(xfeat, xfeat.T, sq.reshape(N, 1), sq.reshape(1, N),
      batch.reshape(N, 1), batch.reshape(1, N))


def _sc_gather(table, idx_flat):
    v, d = table.shape
    b = idx_flat.shape[0]
    nc, ns = 2, 16
    bpw = b // (nc * ns)
    ch = 128
    nch = bpw // ch
    mesh = plsc.VectorSubcoreMesh(core_axis_name="c", subcore_axis_name="s")

    @functools.partial(
        pl.kernel,
        mesh=mesh,
        out_type=jax.ShapeDtypeStruct((b, d), jnp.float32),
        scratch_types=[
            pltpu.VMEM((nch, ch), jnp.int32),
            pltpu.VMEM((ch, d), jnp.float32),
            pltpu.SemaphoreType.DMA,
        ],
        compiler_params=pltpu.CompilerParams(use_tc_tiling_on_sc=False),
    )
    def k(table_hbm, idx_hbm, out_hbm, idx_v, rows_v, sem):
        wid = lax.axis_index("s") * nc + lax.axis_index("c")
        base = wid * nch
        pltpu.sync_copy(idx_hbm.at[pl.ds(base, nch)], idx_v)
        for c in range(nch):
            pltpu.async_copy(table_hbm.at[idx_v.at[c]], rows_v, sem).wait()
            pltpu.sync_copy(rows_v, out_hbm.at[pl.ds((base + c) * ch, ch)])

    return k(table, idx_flat.reshape(b // ch, ch))


def _conv1st_body(xg_ref, xi_ref, w1_ref, b1_ref, h1_ref, st_ref):
    rb = xg_ref.shape[0]
    f = xg_ref.shape[2]
    xj = xg_ref[...].reshape(rb * KNN, f)
    xi = jnp.broadcast_to(xi_ref[...], (rb, KNN, f)).reshape(rb * KNN, f)
    feat = jnp.concatenate([xi, xj - xi], axis=1)
    h = jnp.dot(feat, w1_ref[...], preferred_element_type=jnp.float32) + b1_ref[...]
    h = jnp.maximum(h, 0.0)
    part = jnp.concatenate(
        [jnp.sum(h, axis=0, keepdims=True), jnp.sum(h * h, axis=0, keepdims=True)],
        axis=0,
    )

    @pl.when(pl.program_id(0) == 0)
    def _init():
        st_ref[...] = jnp.zeros_like(st_ref)

    st_ref[...] += part
    h1_ref[...] = h


def _conv1st(xg3, xi3, w1, b1):
    f = xg3.shape[2]
    fo = w1.shape[1]
    r = _CONV_ROWS
    return pl.pallas_call(
        _conv1st_body,
        grid=(N // r,),
        in_specs=[
            pl.BlockSpec((r, KNN, f), lambda i: (i, 0, 0)),
            pl.BlockSpec((r, 1, f), lambda i: (i, 0, 0)),
            pl.BlockSpec((2 * f, fo), lambda i: (0, 0)),
            pl.BlockSpec((1, fo), lambda i: (0, 0)),
        ],
        out_specs=[
            pl.BlockSpec((r * KNN, fo), lambda i: (i, 0)),
            pl.BlockSpec((2, fo), lambda i: (0, 0)),
        ],
        out_shape=[
            jax.ShapeDtypeStruct((N * KNN, fo), jnp.float32),
            jax.ShapeDtypeStruct((2, fo), jnp.float32),
        ],
        compiler_params=pltpu.CompilerParams(dimension_semantics=("arbitrary",)),
    )(xg3, xi3, w1, b1)


def _norm(h, st, g, be, nrows):
    mu = st[0:1, :] * (1.0 / nrows)
    var = st[1:2, :] * (1.0 / nrows) - mu * mu
    return (h - mu) / jnp.sqrt(var + EPS) * g + be


def _convB_body(h1_ref, st1_ref, w2_ref, b2_ref, g1_ref, be1_ref,
                hmax_ref, hmin_ref, st2_ref, *, nk):
    rb = h1_ref.shape[0]
    f = h1_ref.shape[2]
    h1 = _norm(h1_ref[...].reshape(rb * KNN, f), st1_ref[...], g1_ref[...],
               be1_ref[...], nk)
    h2 = jnp.dot(h1, w2_ref[...], preferred_element_type=jnp.float32) + b2_ref[...]
    h2 = jnp.maximum(h2, 0.0)
    part = jnp.concatenate(
        [jnp.sum(h2, axis=0, keepdims=True), jnp.sum(h2 * h2, axis=0, keepdims=True)],
        axis=0,
    )

    @pl.when(pl.program_id(0) == 0)
    def _init():
        st2_ref[...] = jnp.zeros_like(st2_ref)

    st2_ref[...] += part
    h3 = h2.reshape(rb, KNN, h2.shape[1])
    hmax_ref[...] = jnp.max(h3, axis=1)
    hmin_ref[...] = jnp.min(h3, axis=1)


def _convB(h13, st1, w2, b2, g1, be1):
    f = h13.shape[2]
    fo = w2.shape[1]
    r = _CONV_ROWS
    return pl.pallas_call(
        functools.partial(_convB_body, nk=float(N * KNN)),
        grid=(N // r,),
        in_specs=[
            pl.BlockSpec((r, KNN, f), lambda i: (i, 0, 0)),
            pl.BlockSpec((2, f), lambda i: (0, 0)),
            pl.BlockSpec((f, fo), lambda i: (0, 0)),
            pl.BlockSpec((1, fo), lambda i: (0, 0)),
            pl.BlockSpec((1, f), lambda i: (0, 0)),
            pl.BlockSpec((1, f), lambda i: (0, 0)),
        ],
        out_specs=[
            pl.BlockSpec((r, fo), lambda i: (i, 0)),
            pl.BlockSpec((r, fo), lambda i: (i, 0)),
            pl.BlockSpec((2, fo), lambda i: (0, 0)),
        ],
        out_shape=[
            jax.ShapeDtypeStruct((N, fo), jnp.float32),
            jax.ShapeDtypeStruct((N, fo), jnp.float32),
            jax.ShapeDtypeStruct((2, fo), jnp.float32),
        ],
        compiler_params=pltpu.CompilerParams(dimension_semantics=("arbitrary",)),
    )(h13, st1, w2, b2, g1, be1)


def _fin_body(hmax_ref, hmin_ref, st_ref, g_ref, be_ref, x_ref, sq_ref, *, nk):
    g = g_ref[...]
    h = jnp.where(g >= 0.0, hmax_ref[...], hmin_ref[...])
    xv = _norm(h, st_ref[...], g, be_ref[...], nk)
    x_ref[...] = xv
    sq_ref[...] = jnp.sum(xv * xv, axis=1, keepdims=True)


def _fin(hmax, hmin, st, g, be):
    f = hmax.shape[1]
    return pl.pallas_call(
        functools.partial(_fin_body, nk=float(N * KNN)),
        out_shape=(
            jax.ShapeDtypeStruct((N, f), jnp.float32),
            jax.ShapeDtypeStruct((N, 1), jnp.float32),
        ),
    )(hmax, hmin, st, g, be)


def _lin1_body(x1_ref, x2_ref, x3_ref, w_ref, b_ref, o_ref, st_ref):
    hv = jnp.concatenate([x1_ref[...], x2_ref[...], x3_ref[...]], axis=1)
    o = jnp.dot(hv, w_ref[...], preferred_element_type=jnp.float32) + b_ref[...]
    o = jnp.maximum(o, 0.0)
    part = jnp.concatenate(
        [jnp.sum(o, axis=0, keepdims=True), jnp.sum(o * o, axis=0, keepdims=True)],
        axis=0,
    )

    @pl.when(pl.program_id(0) == 0)
    def _init():
        st_ref[...] = jnp.zeros_like(st_ref)

    st_ref[...] += part
    o_ref[...] = o


def _lin1(x1, x2, x3, w, b):
    f = x1.shape[1]
    fo = w.shape[1]
    r = 512
    return pl.pallas_call(
        _lin1_body,
        grid=(N // r,),
        in_specs=[
            pl.BlockSpec((r, f), lambda i: (i, 0)),
            pl.BlockSpec((r, f), lambda i: (i, 0)),
            pl.BlockSpec((r, f), lambda i: (i, 0)),
            pl.BlockSpec((3 * f, fo), lambda i: (0, 0)),
            pl.BlockSpec((1, fo), lambda i: (0, 0)),
        ],
        out_specs=[
            pl.BlockSpec((r, fo), lambda i: (i, 0)),
            pl.BlockSpec((2, fo), lambda i: (0, 0)),
        ],
        out_shape=[
            jax.ShapeDtypeStruct((N, fo), jnp.float32),
            jax.ShapeDtypeStruct((2, fo), jnp.float32),
        ],
        compiler_params=pltpu.CompilerParams(dimension_semantics=("arbitrary",)),
    )(x1, x2, x3, w, b)


def _mid_body(h_ref, st_ref, g_ref, be_ref, w_ref, b_ref, o_ref, st2_ref, *, nrows):
    hv = _norm(h_ref[...], st_ref[...], g_ref[...], be_ref[...], nrows)
    o = jnp.dot(hv, w_ref[...], preferred_element_type=jnp.float32) + b_ref[...]
    o = jnp.maximum(o, 0.0)
    part = jnp.concatenate(
        [jnp.sum(o, axis=0, keepdims=True), jnp.sum(o * o, axis=0, keepdims=True)],
        axis=0,
    )

    @pl.when(pl.program_id(0) == 0)
    def _init():
        st2_ref[...] = jnp.zeros_like(st2_ref)

    st2_ref[...] += part
    o_ref[...] = o


def _mid(h, st, g, be, w, b):
    f = h.shape[1]
    fo = w.shape[1]
    r = 512
    return pl.pallas_call(
        functools.partial(_mid_body, nrows=float(N)),
        grid=(N // r,),
        in_specs=[
            pl.BlockSpec((r, f), lambda i: (i, 0)),
            pl.BlockSpec((2, f), lambda i: (0, 0)),
            pl.BlockSpec((1, f), lambda i: (0, 0)),
            pl.BlockSpec((1, f), lambda i: (0, 0)),
            pl.BlockSpec((f, fo), lambda i: (0, 0)),
            pl.BlockSpec((1, fo), lambda i: (0, 0)),
        ],
        out_specs=[
            pl.BlockSpec((r, fo), lambda i: (i, 0)),
            pl.BlockSpec((2, fo), lambda i: (0, 0)),
        ],
        out_shape=[
            jax.ShapeDtypeStruct((N, fo), jnp.float32),
            jax.ShapeDtypeStruct((2, fo), jnp.float32),
        ],
        compiler_params=pltpu.CompilerParams(dimension_semantics=("arbitrary",)),
    )(h, st, g, be, w, b)


def _out_body(h_ref, st_ref, g_ref, be_ref, w_ref, b_ref, o_ref, *, nrows):
    hv = _norm(h_ref[...], st_ref[...], g_ref[...], be_ref[...], nrows)
    o = jnp.dot(hv, w_ref[...], preferred_element_type=jnp.float32) + b_ref[...]
    m = jnp.max(o, axis=1, keepdims=True)
    e = jnp.exp(o - m)
    lse = jnp.log(jnp.sum(e, axis=1, keepdims=True))
    o_ref[...] = o - m - lse


def _out(h, st, g, be, w, b):
    fo = w.shape[1]
    return pl.pallas_call(
        functools.partial(_out_body, nrows=float(N)),
        out_shape=jax.ShapeDtypeStruct((N, fo), jnp.float32),
    )(h, st, g, be, w, b)


def _conv_stage(xfeat, sq, batch, conv, cdim):
    cp = xfeat.shape[1]
    idx = _knn(xfeat, sq, batch)
    xg = _sc_gather(xfeat, idx.reshape(-1))
    w1 = conv[0]["W"]
    w1r = jnp.concatenate(
        [jnp.pad(w1[:cdim], ((0, cp - cdim), (0, 0))),
         jnp.pad(w1[cdim:], ((0, cp - cdim), (0, 0)))], axis=0)
    h1, st1 = _conv1st(xg.reshape(N, KNN, cp), xfeat.reshape(N, 1, cp),
                       w1r, conv[0]["b"].reshape(1, -1))
    fo = conv[0]["W"].shape[1]
    hmax, hmin, st2 = _convB(
        h1.reshape(N, KNN, fo), st1,
        conv[1]["W"], conv[1]["b"].reshape(1, -1),
        conv[0]["g"].reshape(1, -1), conv[0]["be"].reshape(1, -1),
    )
    return _fin(hmax, hmin, st2, conv[1]["g"].reshape(1, -1),
                conv[1]["be"].reshape(1, -1))


def kernel(x, pos, batch, params):
    batch = batch.astype(jnp.int32)
    x0 = jnp.concatenate([x, pos], axis=1)
    x0p = jnp.pad(x0, ((0, 0), (0, 64 - x0.shape[1])))
    sq0 = _rowsq(x0p)

    x1, sq1 = _conv_stage(x0p, sq0, batch, params["conv1"], x0.shape[1])
    x2, sq2 = _conv_stage(x1, sq1, batch, params["conv2"], x1.shape[1])
    x3, _ = _conv_stage(x2, sq2, batch, params["conv3"], x2.shape[1])

    l1 = params["lin1"][0]
    m1 = params["mlp1"][0]
    m2 = params["mlp2"][0]
    h, st = _lin1(x1, x2, x3, l1["W"], l1["b"].reshape(1, -1))
    h, st = _mid(h, st, l1["g"].reshape(1, -1), l1["be"].reshape(1, -1),
                 m1["W"], m1["b"].reshape(1, -1))
    h, st = _mid(h, st, m1["g"].reshape(1, -1), m1["be"].reshape(1, -1),
                 m2["W"], m2["b"].reshape(1, -1))
    return _out(h, st, m2["g"].reshape(1, -1), m2["be"].reshape(1, -1),
                params["out_W"], params["out_b"].reshape(1, -1))
